# baseline (speedup 1.0000x reference)
import functools

import jax
import jax.numpy as jnp
from jax import lax
from jax.experimental import pallas as pl
from jax.experimental.pallas import tpu as pltpu
from jax.experimental.pallas import tpu_sc as plsc

_sc_kernel = functools.partial(
    pl.kernel,
    compiler_params=pltpu.CompilerParams(needs_layout_passes=False))

N = 10000
N2 = 10240
NC = 2500
NC2 = 2560
E = 320000
EC = 80000
C = 128
NCORES = 2
NSUB = 16
NW = NCORES * NSUB
CH = 80
F32 = jnp.float32
I32 = jnp.int32


def _worker_id():
    return lax.axis_index("s") * NCORES + lax.axis_index("c")


def _zero_vmem_2d(ref, nrows):
    z = jnp.zeros((16,), F32)

    def body(i, _):
        ref[i // (C // 16), pl.ds((i % (C // 16)) * 16, 16)] = z
        return 0

    lax.fori_loop(0, nrows * (C // 16), body, 0)


def _zero_vmem_1d(ref, n):
    z = jnp.zeros((16,), F32)

    def body(i, _):
        ref[pl.ds(i * 16, 16)] = z
        return 0

    lax.fori_loop(0, n // 16, body, 0)


def _edge_pass(table, src, dst, acc_rows, with_counts, K):
    n_edges = dst.shape[0]
    n_chunks = n_edges // CH
    trips = n_chunks // NW
    assert n_edges % CH == 0 and n_chunks % NW == 0 and trips % K == 0
    rpt = acc_rows // NSUB
    nfull, rem = divmod(rpt, CH)
    assert rem == 0

    out_type = [jax.ShapeDtypeStruct((NCORES, acc_rows, C), F32)]
    scratch = [pltpu.VMEM_SHARED((acc_rows, C), F32)]
    scratch += [pltpu.VMEM((CH, C), F32) for _ in range(K)]
    scratch += [pltpu.VMEM((CH,), I32) for _ in range(2 * K)]
    scratch += [pltpu.SemaphoreType.DMA for _ in range(2 * K + 1)]
    if with_counts:
        out_type.append(jax.ShapeDtypeStruct((NCORES * acc_rows,), F32))
        out_type.append(jax.ShapeDtypeStruct((NW * acc_rows,), F32))
        scratch.append(pltpu.VMEM((acc_rows,), F32))
        scratch.append(pltpu.VMEM((NSUB * rpt,), F32))

    def body(*refs):
        it = iter(refs)
        table_h, src_h, dst_h = next(it), next(it), next(it)
        sums_h = next(it)
        cnts_h = next(it) if with_counts else None
        cparts_h = next(it) if with_counts else None
        acc = next(it)
        rows = [next(it) for _ in range(K)]
        srcv = [next(it) for _ in range(K)]
        dstv = [next(it) for _ in range(K)]
        isem = [next(it) for _ in range(K)]
        gsem = [next(it) for _ in range(K)]
        ssem = next(it)
        if with_counts:
            cntv, redv = next(it), next(it)

        cidx = lax.axis_index("c")
        sidx = lax.axis_index("s")
        w = _worker_id()

        _zero_vmem_2d(rows[0], CH)
        base = sidx * rpt
        for i in range(nfull):
            pltpu.sync_copy(rows[0], acc.at[pl.ds(base + i * CH, CH)])
        if with_counts:
            _zero_vmem_1d(cntv, acc_rows)
        plsc.subcore_barrier()

        ones16 = jnp.ones((16,), F32)

        def off_of(j):
            return (j * NW + w) * CH

        def do_counts(dv):
            if with_counts:
                for g in range(CH // 16):
                    d16 = dv[pl.ds(g * 16, 16)]
                    plsc.addupdate_scatter(cntv, (d16,), ones16)

        if K == 1:
            def trip(j, _):
                off = off_of(j)
                pltpu.sync_copy(src_h.at[pl.ds(off, CH)], srcv[0])
                pltpu.sync_copy(dst_h.at[pl.ds(off, CH)], dstv[0])
                pltpu.async_copy(table_h.at[srcv[0]], rows[0], gsem[0]).wait()
                pltpu.sync_copy(rows[0], acc.at[dstv[0]], add=True)
                do_counts(dstv[0])
                return 0

            lax.fori_loop(0, trips, trip, 0)

        def titer(t, _):
            j0 = t * K
            idescs = []
            for b in range(K):
                off = off_of(j0 + b)
                idescs.append(pltpu.async_copy(
                    src_h.at[pl.ds(off, CH)], srcv[b], isem[b]))
                idescs.append(pltpu.async_copy(
                    dst_h.at[pl.ds(off, CH)], dstv[b], isem[b]))
            gdescs = []
            for b in range(K):
                idescs[2 * b].wait()
                idescs[2 * b + 1].wait()
                gdescs.append(pltpu.async_copy(
                    table_h.at[srcv[b]], rows[b], gsem[b]))
            sdescs = []
            for b in range(K):
                gdescs[b].wait()
                sdescs.append(pltpu.async_copy(
                    rows[b], acc.at[dstv[b]], ssem, add=True))
                do_counts(dstv[b])
            for d in sdescs:
                d.wait()
            return 0

        if K > 1:
            lax.fori_loop(0, trips // K, titer, 0)

        if with_counts:
            pltpu.sync_copy(
                cntv,
                cparts_h.at[pl.ds((cidx * NSUB + sidx) * acc_rows, acc_rows)])
            plsc.subcore_barrier()
            rdescs = [pltpu.async_copy(
                cparts_h.at[pl.ds((cidx * NSUB + p) * acc_rows + base, rpt)],
                redv.at[pl.ds(p * rpt, rpt)], ssem) for p in range(NSUB)]
            for d in rdescs:
                d.wait()

            def redloop(i, _):
                tot = redv[pl.ds(i * 16, 16)]
                for p in range(1, NSUB):
                    tot = tot + redv[pl.ds(p * rpt + i * 16, 16)]
                cntv[pl.ds(i * 16, 16)] = tot
                return 0

            lax.fori_loop(0, rpt // 16, redloop, 0)
            pltpu.sync_copy(cntv.at[pl.ds(0, rpt)],
                            cnts_h.at[pl.ds(cidx * acc_rows + base, rpt)])

        plsc.subcore_barrier()
        pltpu.sync_copy(acc.at[pl.ds(base, rpt)],
                        sums_h.at[cidx, pl.ds(base, rpt)])

    mesh = plsc.VectorSubcoreMesh(core_axis_name="c", subcore_axis_name="s")
    fn = _sc_kernel(body, out_type=tuple(out_type), mesh=mesh,
                    scratch_types=tuple(scratch))
    return fn(table, src, dst)


def _edge_pass_r1(table, src, dst, acc_rows, with_counts):
    n_edges = dst.shape[0]
    assert n_edges % CH == 0
    n_chunks = n_edges // CH
    trips = -(-n_chunks // NW)
    rpt = acc_rows // NSUB
    nfull, rem = divmod(rpt, CH)

    out_type = [jax.ShapeDtypeStruct((NCORES, acc_rows, C), F32)]
    scratch = [
        pltpu.VMEM_SHARED((acc_rows, C), F32),
        pltpu.VMEM((CH, C), F32),
        pltpu.VMEM((CH,), I32),
        pltpu.VMEM((CH,), I32),
        pltpu.SemaphoreType.DMA,
    ]
    if with_counts:
        out_type.append(jax.ShapeDtypeStruct((NCORES * acc_rows,), F32))
        scratch.append(pltpu.VMEM_SHARED((acc_rows,), F32))
        scratch.append(pltpu.VMEM((acc_rows,), F32))
        scratch.append(pltpu.VMEM((CH,), I32))

    def body(*refs):
        it = iter(refs)
        table_h, src_h, dst_h = next(it), next(it), next(it)
        sums_h = next(it)
        cnts_h = next(it) if with_counts else None
        acc, rows, dstv, srcv, sem = (next(it), next(it), next(it), next(it),
                                      next(it))
        if with_counts:
            cacc, cntv, iotav = next(it), next(it), next(it)

        cidx = lax.axis_index("c")
        sidx = lax.axis_index("s")
        w = _worker_id()

        _zero_vmem_2d(rows, CH)
        base = sidx * rpt
        for i in range(nfull):
            pltpu.sync_copy(rows, acc.at[pl.ds(base + i * CH, CH)])
        if rem:
            pltpu.sync_copy(rows.at[pl.ds(0, rem)],
                            acc.at[pl.ds(base + nfull * CH, rem)])
        if with_counts:
            _zero_vmem_1d(cntv, acc_rows)
            for i in range(nfull):
                pltpu.sync_copy(cntv.at[pl.ds(0, CH)],
                                cacc.at[pl.ds(base + i * CH, CH)])
            if rem:
                pltpu.sync_copy(cntv.at[pl.ds(0, rem)],
                                cacc.at[pl.ds(base + nfull * CH, rem)])
        plsc.subcore_barrier()

        ones16 = jnp.ones((16,), F32)

        def trip(j, _):
            k = j * NW + w

            @pl.when(k < n_chunks)
            def _():
                off = k * CH
                pltpu.sync_copy(dst_h.at[pl.ds(off, CH)], dstv)
                pltpu.sync_copy(src_h.at[pl.ds(off, CH)], srcv)
                pltpu.async_copy(table_h.at[srcv], rows, sem).wait()
                pltpu.sync_copy(rows, acc.at[dstv], add=True)
                if with_counts:
                    for g in range(CH // 16):
                        d16 = dstv[pl.ds(g * 16, 16)]
                        plsc.addupdate_scatter(cntv, (d16,), ones16)

            return 0

        lax.fori_loop(0, trips, trip, 0)

        if with_counts:
            iota16 = lax.iota(I32, 16)
            for g in range(CH // 16):
                iotav[pl.ds(g * 16, 16)] = iota16 + g * 16

            def cmerge(j, _):
                pltpu.sync_copy(cntv.at[pl.ds(j * CH, CH)],
                                cacc.at[iotav], add=True)

                def bump(g, _):
                    v = iotav[pl.ds(g * 16, 16)]
                    iotav[pl.ds(g * 16, 16)] = v + CH
                    return 0

                lax.fori_loop(0, CH // 16, bump, 0)
                return 0

            lax.fori_loop(0, acc_rows // CH, cmerge, 0)

        plsc.subcore_barrier()
        pltpu.sync_copy(acc.at[pl.ds(base, rpt)],
                        sums_h.at[cidx, pl.ds(base, rpt)])
        if with_counts:
            pltpu.sync_copy(cacc.at[pl.ds(base, rpt)], cntv.at[pl.ds(0, rpt)])
            pltpu.sync_copy(cntv.at[pl.ds(0, rpt)],
                            cnts_h.at[pl.ds(cidx * acc_rows + base, rpt)])

    mesh = plsc.VectorSubcoreMesh(core_axis_name="c", subcore_axis_name="s")
    fn = _sc_kernel(body, out_type=tuple(out_type), mesh=mesh,
                    scratch_types=tuple(scratch))
    return fn(table, src, dst)


def _pool_pass(sums, cnts, pool_idx):
    n_chunks = N2 // CH
    trips = n_chunks // NW
    assert n_chunks % NW == 0
    rpt = NC2 // NSUB
    nfull, rem = divmod(rpt, CH)

    out_type = (jax.ShapeDtypeStruct((NCORES, NC2, C), F32),
                jax.ShapeDtypeStruct((NCORES * NC2,), F32))
    scratch = (
        pltpu.VMEM_SHARED((NC2, C), F32),
        pltpu.VMEM_SHARED((NC2,), F32),
        pltpu.VMEM((CH, C), F32),
        pltpu.VMEM((CH, C), F32),
        pltpu.VMEM((CH, C), F32),
        pltpu.VMEM((CH,), F32),
        pltpu.VMEM((CH,), F32),
        pltpu.VMEM((CH,), I32),
        pltpu.VMEM((NC2,), F32),
        pltpu.VMEM((CH,), I32),
        pltpu.SemaphoreType.DMA,
    )

    def body(s_h, c_h, pool_h, psums_h, pcnts_h,
             acc, cacc, rows0, rows1, hbuf, c0v, c1v, dstv, cntv, iotav, sem):
        cidx = lax.axis_index("c")
        sidx = lax.axis_index("s")
        w = _worker_id()

        _zero_vmem_2d(hbuf, CH)
        base = sidx * rpt
        for i in range(nfull):
            pltpu.sync_copy(hbuf, acc.at[pl.ds(base + i * CH, CH)])
        if rem:
            pltpu.sync_copy(hbuf.at[pl.ds(0, rem)],
                            acc.at[pl.ds(base + nfull * CH, rem)])
        _zero_vmem_1d(cntv, NC2)
        for i in range(nfull):
            pltpu.sync_copy(cntv.at[pl.ds(0, CH)],
                            cacc.at[pl.ds(base + i * CH, CH)])
        if rem:
            pltpu.sync_copy(cntv.at[pl.ds(0, rem)],
                            cacc.at[pl.ds(base + nfull * CH, rem)])
        plsc.subcore_barrier()

        ones16 = jnp.ones((16,), F32)
        one16 = jnp.ones((16,), F32)

        def trip(j, _):
            off = (j * NW + w) * CH
            pltpu.sync_copy(s_h.at[0, pl.ds(off, CH)], rows0)
            pltpu.sync_copy(s_h.at[1, pl.ds(off, CH)], rows1)
            pltpu.sync_copy(c_h.at[pl.ds(off, CH)], c0v)
            pltpu.sync_copy(c_h.at[pl.ds(N2 + off, CH)], c1v)
            pltpu.sync_copy(pool_h.at[pl.ds(off, CH)], dstv)

            def row(r, _):
                ridx = jnp.full((16,), r, I32)
                d = (plsc.load_gather(c0v, (ridx,))
                     + plsc.load_gather(c1v, (ridx,)))
                rcp = one16 / jnp.maximum(d, 1.0)
                for f in range(C // 16):
                    v = (rows0[r, pl.ds(f * 16, 16)]
                         + rows1[r, pl.ds(f * 16, 16)]) * rcp
                    hbuf[r, pl.ds(f * 16, 16)] = jnp.maximum(v, 0.0)
                return 0

            lax.fori_loop(0, CH, row, 0)
            pltpu.sync_copy(hbuf, acc.at[dstv], add=True)
            for g in range(CH // 16):
                d16 = dstv[pl.ds(g * 16, 16)]
                plsc.addupdate_scatter(cntv, (d16,), ones16)

            return 0

        lax.fori_loop(0, trips, trip, 0)

        iota16 = lax.iota(I32, 16)
        for g in range(CH // 16):
            iotav[pl.ds(g * 16, 16)] = iota16 + g * 16

        def cmerge(j, _):
            pltpu.sync_copy(cntv.at[pl.ds(j * CH, CH)], cacc.at[iotav], add=True)

            def bump(g, _):
                v = iotav[pl.ds(g * 16, 16)]
                iotav[pl.ds(g * 16, 16)] = v + CH
                return 0

            lax.fori_loop(0, CH // 16, bump, 0)
            return 0

        lax.fori_loop(0, NC2 // CH, cmerge, 0)

        plsc.subcore_barrier()
        pltpu.sync_copy(acc.at[pl.ds(base, rpt)],
                        psums_h.at[cidx, pl.ds(base, rpt)])
        pltpu.sync_copy(cacc.at[pl.ds(base, rpt)], cntv.at[pl.ds(0, rpt)])
        pltpu.sync_copy(cntv.at[pl.ds(0, rpt)],
                        pcnts_h.at[pl.ds(cidx * NC2 + base, rpt)])

    mesh = plsc.VectorSubcoreMesh(core_axis_name="c", subcore_axis_name="s")
    fn = _sc_kernel(body, out_type=out_type, mesh=mesh, scratch_types=scratch)
    return fn(sums, cnts, pool_idx)


def _gather_rows(table, idx):
    n_chunks = N // CH
    trips = -(-n_chunks // NW)

    def body(table_h, idx_h, out_h, idxv, rows, sem):
        w = _worker_id()

        def trip(j, _):
            k = j * NW + w

            @pl.when(k < n_chunks)
            def _():
                off = k * CH
                pltpu.sync_copy(idx_h.at[pl.ds(off, CH)], idxv)
                pltpu.async_copy(table_h.at[idxv], rows, sem).wait()
                pltpu.sync_copy(rows, out_h.at[pl.ds(off, CH)])

            return 0

        lax.fori_loop(0, trips, trip, 0)

    mesh = plsc.VectorSubcoreMesh(core_axis_name="c", subcore_axis_name="s")
    fn = _sc_kernel(body,
                    out_type=jax.ShapeDtypeStruct((N, C), F32),
                    mesh=mesh,
                    scratch_types=(pltpu.VMEM((CH,), I32),
                                   pltpu.VMEM((CH, C), F32),
                                   pltpu.SemaphoreType.DMA))
    return fn(table, idx)


def _tc_call(fn, out_type, *args):
    return pl.pallas_call(fn, out_shape=out_type)(*args)


def _k_matmul(x, w):
    def body(x_ref, w_ref, o_ref):
        o_ref[...] = jnp.dot(x_ref[...], w_ref[...],
                             preferred_element_type=F32)

    return _tc_call(body, jax.ShapeDtypeStruct((x.shape[0], w.shape[1]), F32),
                    x, w)


def _k_mean_mm(sums, cnt3, w, relu):
    def body(s_ref, c_ref, w_ref, x_ref, y_ref):
        s = s_ref[0] + s_ref[1]
        d = jnp.maximum(c_ref[0] + c_ref[1], 1.0)
        x = s / d
        if relu:
            x = jnp.maximum(x, 0.0)
        x_ref[...] = x
        y_ref[...] = jnp.dot(x, w_ref[...], preferred_element_type=F32)

    out = (jax.ShapeDtypeStruct((NC2, C), F32),
           jax.ShapeDtypeStruct((NC2, C), F32))
    return _tc_call(body, out, sums, cnt3, w)


def _k_skip_merge(csums, cnt3, hc2, wm, pmat):
    def body(s_ref, c_ref, h_ref, wm_ref, p_ref, m_ref, ps_ref):
        d = jnp.maximum(c_ref[0] + c_ref[1], 1.0)
        s2 = (s_ref[0] + s_ref[1]) / d
        hc2 = h_ref[...]
        skip = jnp.maximum(s2 + hc2, 0.0)
        m_ref[...] = (jnp.dot(hc2, wm_ref[:C], preferred_element_type=F32)
                      + jnp.dot(skip, wm_ref[C:], preferred_element_type=F32))
        pa = jnp.dot(hc2, p_ref[...], preferred_element_type=F32)
        pb = jnp.dot(skip, p_ref[...], preferred_element_type=F32)
        ps_ref[...] = jnp.concatenate([pa, pb], axis=-1)

    out = (jax.ShapeDtypeStruct((NC2, C), F32),
           jax.ShapeDtypeStruct((NC2, C), F32))
    return _tc_call(body, out, csums, cnt3, hc2, wm, pmat)


def _k_final_mm(dsums, cnt3, pairsum, wup):
    def body(s_ref, c_ref, p_ref, w_ref, u_ref):
        d = jnp.maximum(c_ref[0] + c_ref[1], 1.0)
        merge = jnp.maximum((s_ref[0] + s_ref[1]) / d, 0.0)
        rf = merge + p_ref[...]
        u_ref[...] = jnp.maximum(
            jnp.dot(rf, w_ref[...], preferred_element_type=F32), 0.0)

    return _tc_call(body, jax.ShapeDtypeStruct((NC2, C), F32),
                    dsums, cnt3, pairsum, wup)


@jax.jit
def kernel(point_feat, edge_index, coarse_edge_index, pool_idx,
           W0, W1, Ws1, Ws2, Wm, Wup):
    EP = 322560
    ECP = 81920
    src = jnp.concatenate([edge_index[0], jnp.zeros((EP - E,), I32)])
    dst = jnp.concatenate([edge_index[1], jnp.full((EP - E,), N, I32)])
    csrc = jnp.concatenate([coarse_edge_index[0],
                            jnp.zeros((ECP - EC,), I32)])
    cdst = jnp.concatenate([coarse_edge_index[1],
                            jnp.full((ECP - EC,), NC, I32)])
    pool_pad = jnp.concatenate([pool_idx, jnp.full((N2 - N,), NC, I32)])
    pmat = jnp.repeat(jnp.eye(64, dtype=F32), 2, axis=0)

    p0 = _k_matmul(point_feat, W0)
    fsums, fcnts = _edge_pass_r1(p0, src, dst, N2, True)
    psums, pcnts = _pool_pass(fsums, fcnts, pool_pad)
    pcnt3 = pcnts.reshape(NCORES, NC2, 1)
    hc, h1 = _k_mean_mm(psums, pcnt3, W1, relu=False)
    asums, ccnts = _edge_pass_r1(h1, csrc, cdst, NC2, True)
    ccnt3 = ccnts.reshape(NCORES, NC2, 1)
    hc2, s1 = _k_mean_mm(asums, ccnt3, Ws1, relu=True)
    bsums = _edge_pass_r1(s1, csrc, cdst, NC2, False)[0]
    s, s2m = _k_mean_mm(bsums, ccnt3, Ws2, relu=True)
    csums = _edge_pass_r1(s2m, csrc, cdst, NC2, False)[0]
    m, pairsum = _k_skip_merge(csums, ccnt3, hc2, Wm, pmat)
    dsums = _edge_pass_r1(m, csrc, cdst, NC2, False)[0]
    u = _k_final_mm(dsums, ccnt3, pairsum, Wup)
    return _gather_rows(u, pool_idx)


# speedup vs baseline: 1.0000x; 1.0000x over previous
import functools

import jax
import jax.numpy as jnp
from jax import lax
from jax.experimental import pallas as pl
from jax.experimental.pallas import tpu as pltpu
from jax.experimental.pallas import tpu_sc as plsc

_sc_kernel = functools.partial(
    pl.kernel,
    compiler_params=pltpu.CompilerParams(needs_layout_passes=False))

N = 10000
N2 = 10240
NC = 2500
NC2 = 2560
E = 320000
EC = 80000
C = 128
NCORES = 2
NSUB = 16
NW = NCORES * NSUB
CH = 80
F32 = jnp.float32
I32 = jnp.int32


def _worker_id():
    return lax.axis_index("s") * NCORES + lax.axis_index("c")


def _zero_vmem_2d(ref, nrows):
    z = jnp.zeros((16,), F32)

    def body(i, _):
        ref[i // (C // 16), pl.ds((i % (C // 16)) * 16, 16)] = z
        return 0

    lax.fori_loop(0, nrows * (C // 16), body, 0)


def _zero_vmem_1d(ref, n):
    z = jnp.zeros((16,), F32)

    def body(i, _):
        ref[pl.ds(i * 16, 16)] = z
        return 0

    lax.fori_loop(0, n // 16, body, 0)


def _edge_pass(table, src, dst, acc_rows, with_counts, K):
    n_edges = dst.shape[0]
    n_chunks = n_edges // CH
    trips = n_chunks // NW
    assert n_edges % CH == 0 and n_chunks % NW == 0 and trips % K == 0
    rpt = acc_rows // NSUB
    nfull, rem = divmod(rpt, CH)
    assert rem == 0

    out_type = [jax.ShapeDtypeStruct((NCORES, acc_rows, C), F32)]
    scratch = [pltpu.VMEM_SHARED((acc_rows, C), F32)]
    scratch += [pltpu.VMEM((CH, C), F32) for _ in range(K)]
    scratch += [pltpu.VMEM((CH,), I32) for _ in range(2 * K)]
    scratch += [pltpu.SemaphoreType.DMA for _ in range(2 * K + 1)]
    if with_counts:
        out_type.append(jax.ShapeDtypeStruct((NCORES * acc_rows,), F32))
        out_type.append(jax.ShapeDtypeStruct((NW * acc_rows,), F32))
        scratch.append(pltpu.VMEM((acc_rows,), F32))
        scratch.append(pltpu.VMEM((NSUB * rpt,), F32))

    def body(*refs):
        it = iter(refs)
        table_h, src_h, dst_h = next(it), next(it), next(it)
        sums_h = next(it)
        cnts_h = next(it) if with_counts else None
        cparts_h = next(it) if with_counts else None
        acc = next(it)
        rows = [next(it) for _ in range(K)]
        srcv = [next(it) for _ in range(K)]
        dstv = [next(it) for _ in range(K)]
        isem = [next(it) for _ in range(K)]
        gsem = [next(it) for _ in range(K)]
        ssem = next(it)
        if with_counts:
            cntv, redv = next(it), next(it)

        cidx = lax.axis_index("c")
        sidx = lax.axis_index("s")
        w = _worker_id()

        _zero_vmem_2d(rows[0], CH)
        base = sidx * rpt
        for i in range(nfull):
            pltpu.sync_copy(rows[0], acc.at[pl.ds(base + i * CH, CH)])
        if with_counts:
            _zero_vmem_1d(cntv, acc_rows)
        plsc.subcore_barrier()

        ones16 = jnp.ones((16,), F32)

        def off_of(j):
            return (j * NW + w) * CH

        def do_counts(dv):
            if with_counts:
                for g in range(CH // 16):
                    d16 = dv[pl.ds(g * 16, 16)]
                    plsc.addupdate_scatter(cntv, (d16,), ones16)

        if K == 1:
            def trip(j, _):
                off = off_of(j)
                pltpu.sync_copy(src_h.at[pl.ds(off, CH)], srcv[0])
                pltpu.sync_copy(dst_h.at[pl.ds(off, CH)], dstv[0])
                pltpu.async_copy(table_h.at[srcv[0]], rows[0], gsem[0]).wait()
                pltpu.sync_copy(rows[0], acc.at[dstv[0]], add=True)
                do_counts(dstv[0])
                return 0

            lax.fori_loop(0, trips, trip, 0)

        def titer(t, _):
            j0 = t * K
            idescs = []
            for b in range(K):
                off = off_of(j0 + b)
                idescs.append(pltpu.async_copy(
                    src_h.at[pl.ds(off, CH)], srcv[b], isem[b]))
                idescs.append(pltpu.async_copy(
                    dst_h.at[pl.ds(off, CH)], dstv[b], isem[b]))
            gdescs = []
            for b in range(K):
                idescs[2 * b].wait()
                idescs[2 * b + 1].wait()
                gdescs.append(pltpu.async_copy(
                    table_h.at[srcv[b]], rows[b], gsem[b]))
            sdescs = []
            for b in range(K):
                gdescs[b].wait()
                sdescs.append(pltpu.async_copy(
                    rows[b], acc.at[dstv[b]], ssem, add=True))
                do_counts(dstv[b])
            for d in sdescs:
                d.wait()
            return 0

        if K > 1:
            lax.fori_loop(0, trips // K, titer, 0)

        if with_counts:
            pltpu.sync_copy(
                cntv,
                cparts_h.at[pl.ds((cidx * NSUB + sidx) * acc_rows, acc_rows)])
            plsc.subcore_barrier()
            rdescs = [pltpu.async_copy(
                cparts_h.at[pl.ds((cidx * NSUB + p) * acc_rows + base, rpt)],
                redv.at[pl.ds(p * rpt, rpt)], ssem) for p in range(NSUB)]
            for d in rdescs:
                d.wait()

            def redloop(i, _):
                tot = redv[pl.ds(i * 16, 16)]
                for p in range(1, NSUB):
                    tot = tot + redv[pl.ds(p * rpt + i * 16, 16)]
                cntv[pl.ds(i * 16, 16)] = tot
                return 0

            lax.fori_loop(0, rpt // 16, redloop, 0)
            pltpu.sync_copy(cntv.at[pl.ds(0, rpt)],
                            cnts_h.at[pl.ds(cidx * acc_rows + base, rpt)])

        plsc.subcore_barrier()
        pltpu.sync_copy(acc.at[pl.ds(base, rpt)],
                        sums_h.at[cidx, pl.ds(base, rpt)])

    mesh = plsc.VectorSubcoreMesh(core_axis_name="c", subcore_axis_name="s")
    fn = _sc_kernel(body, out_type=tuple(out_type), mesh=mesh,
                    scratch_types=tuple(scratch))
    return fn(table, src, dst)


def _edge_pass_r1(table, src, dst, acc_rows, with_counts):
    n_edges = dst.shape[0]
    assert n_edges % CH == 0
    n_chunks = n_edges // CH
    trips = -(-n_chunks // NW)
    rpt = acc_rows // NSUB
    nfull, rem = divmod(rpt, CH)

    out_type = [jax.ShapeDtypeStruct((NCORES, acc_rows, C), F32)]
    scratch = [
        pltpu.VMEM_SHARED((acc_rows, C), F32),
        pltpu.VMEM((CH, C), F32),
        pltpu.VMEM((CH,), I32),
        pltpu.VMEM((CH,), I32),
        pltpu.SemaphoreType.DMA,
    ]
    if with_counts:
        out_type.append(jax.ShapeDtypeStruct((NCORES * acc_rows,), F32))
        scratch.append(pltpu.VMEM_SHARED((acc_rows,), F32))
        scratch.append(pltpu.VMEM((acc_rows,), F32))
        scratch.append(pltpu.VMEM((CH,), I32))

    def body(*refs):
        it = iter(refs)
        table_h, src_h, dst_h = next(it), next(it), next(it)
        sums_h = next(it)
        cnts_h = next(it) if with_counts else None
        acc, rows, dstv, srcv, sem = (next(it), next(it), next(it), next(it),
                                      next(it))
        if with_counts:
            cacc, cntv, iotav = next(it), next(it), next(it)

        cidx = lax.axis_index("c")
        sidx = lax.axis_index("s")
        w = _worker_id()

        _zero_vmem_2d(rows, CH)
        base = sidx * rpt
        for i in range(nfull):
            pltpu.sync_copy(rows, acc.at[pl.ds(base + i * CH, CH)])
        if rem:
            pltpu.sync_copy(rows.at[pl.ds(0, rem)],
                            acc.at[pl.ds(base + nfull * CH, rem)])
        if with_counts:
            _zero_vmem_1d(cntv, acc_rows)
            for i in range(nfull):
                pltpu.sync_copy(cntv.at[pl.ds(0, CH)],
                                cacc.at[pl.ds(base + i * CH, CH)])
            if rem:
                pltpu.sync_copy(cntv.at[pl.ds(0, rem)],
                                cacc.at[pl.ds(base + nfull * CH, rem)])
        plsc.subcore_barrier()

        ones16 = jnp.ones((16,), F32)

        def trip(j, _):
            k = j * NW + w

            @pl.when(k < n_chunks)
            def _():
                off = k * CH
                pltpu.sync_copy(dst_h.at[pl.ds(off, CH)], dstv)
                pltpu.sync_copy(src_h.at[pl.ds(off, CH)], srcv)
                pltpu.async_copy(table_h.at[srcv], rows, sem).wait()
                pltpu.sync_copy(rows, acc.at[dstv], add=True)
                if with_counts:
                    for g in range(CH // 16):
                        d16 = dstv[pl.ds(g * 16, 16)]
                        plsc.addupdate_scatter(cntv, (d16,), ones16)

            return 0

        lax.fori_loop(0, trips, trip, 0)

        if with_counts:
            iota16 = lax.iota(I32, 16)
            for g in range(CH // 16):
                iotav[pl.ds(g * 16, 16)] = iota16 + g * 16

            def cmerge(j, _):
                pltpu.sync_copy(cntv.at[pl.ds(j * CH, CH)],
                                cacc.at[iotav], add=True)

                def bump(g, _):
                    v = iotav[pl.ds(g * 16, 16)]
                    iotav[pl.ds(g * 16, 16)] = v + CH
                    return 0

                lax.fori_loop(0, CH // 16, bump, 0)
                return 0

            lax.fori_loop(0, acc_rows // CH, cmerge, 0)

        plsc.subcore_barrier()
        pltpu.sync_copy(acc.at[pl.ds(base, rpt)],
                        sums_h.at[cidx, pl.ds(base, rpt)])
        if with_counts:
            pltpu.sync_copy(cacc.at[pl.ds(base, rpt)], cntv.at[pl.ds(0, rpt)])
            pltpu.sync_copy(cntv.at[pl.ds(0, rpt)],
                            cnts_h.at[pl.ds(cidx * acc_rows + base, rpt)])

    mesh = plsc.VectorSubcoreMesh(core_axis_name="c", subcore_axis_name="s")
    fn = _sc_kernel(body, out_type=tuple(out_type), mesh=mesh,
                    scratch_types=tuple(scratch))
    return fn(table, src, dst)


def _pool_pass(sums, cnts, pool_idx):
    n_chunks = N2 // CH
    trips = n_chunks // NW
    assert n_chunks % NW == 0
    rpt = NC2 // NSUB
    nfull, rem = divmod(rpt, CH)

    out_type = (jax.ShapeDtypeStruct((NCORES, NC2, C), F32),
                jax.ShapeDtypeStruct((NCORES * NC2,), F32))
    scratch = (
        pltpu.VMEM_SHARED((NC2, C), F32),
        pltpu.VMEM_SHARED((NC2,), F32),
        pltpu.VMEM((CH, C), F32),
        pltpu.VMEM((CH, C), F32),
        pltpu.VMEM((CH, C), F32),
        pltpu.VMEM((CH,), F32),
        pltpu.VMEM((CH,), F32),
        pltpu.VMEM((CH,), I32),
        pltpu.VMEM((NC2,), F32),
        pltpu.VMEM((CH,), I32),
        pltpu.SemaphoreType.DMA,
    )

    def body(s_h, c_h, pool_h, psums_h, pcnts_h,
             acc, cacc, rows0, rows1, hbuf, c0v, c1v, dstv, cntv, iotav, sem):
        cidx = lax.axis_index("c")
        sidx = lax.axis_index("s")
        w = _worker_id()

        _zero_vmem_2d(hbuf, CH)
        base = sidx * rpt
        for i in range(nfull):
            pltpu.sync_copy(hbuf, acc.at[pl.ds(base + i * CH, CH)])
        if rem:
            pltpu.sync_copy(hbuf.at[pl.ds(0, rem)],
                            acc.at[pl.ds(base + nfull * CH, rem)])
        _zero_vmem_1d(cntv, NC2)
        for i in range(nfull):
            pltpu.sync_copy(cntv.at[pl.ds(0, CH)],
                            cacc.at[pl.ds(base + i * CH, CH)])
        if rem:
            pltpu.sync_copy(cntv.at[pl.ds(0, rem)],
                            cacc.at[pl.ds(base + nfull * CH, rem)])
        plsc.subcore_barrier()

        ones16 = jnp.ones((16,), F32)
        one16 = jnp.ones((16,), F32)

        def trip(j, _):
            off = (j * NW + w) * CH
            pltpu.sync_copy(s_h.at[0, pl.ds(off, CH)], rows0)
            pltpu.sync_copy(s_h.at[1, pl.ds(off, CH)], rows1)
            pltpu.sync_copy(c_h.at[pl.ds(off, CH)], c0v)
            pltpu.sync_copy(c_h.at[pl.ds(N2 + off, CH)], c1v)
            pltpu.sync_copy(pool_h.at[pl.ds(off, CH)], dstv)

            def row(r, _):
                ridx = jnp.full((16,), r, I32)
                d = (plsc.load_gather(c0v, (ridx,))
                     + plsc.load_gather(c1v, (ridx,)))
                rcp = one16 / jnp.maximum(d, 1.0)
                for f in range(C // 16):
                    v = (rows0[r, pl.ds(f * 16, 16)]
                         + rows1[r, pl.ds(f * 16, 16)]) * rcp
                    hbuf[r, pl.ds(f * 16, 16)] = jnp.maximum(v, 0.0)
                return 0

            lax.fori_loop(0, CH, row, 0)
            pltpu.sync_copy(hbuf, acc.at[dstv], add=True)
            for g in range(CH // 16):
                d16 = dstv[pl.ds(g * 16, 16)]
                plsc.addupdate_scatter(cntv, (d16,), ones16)

            return 0

        lax.fori_loop(0, trips, trip, 0)

        iota16 = lax.iota(I32, 16)
        for g in range(CH // 16):
            iotav[pl.ds(g * 16, 16)] = iota16 + g * 16

        def cmerge(j, _):
            pltpu.sync_copy(cntv.at[pl.ds(j * CH, CH)], cacc.at[iotav], add=True)

            def bump(g, _):
                v = iotav[pl.ds(g * 16, 16)]
                iotav[pl.ds(g * 16, 16)] = v + CH
                return 0

            lax.fori_loop(0, CH // 16, bump, 0)
            return 0

        lax.fori_loop(0, NC2 // CH, cmerge, 0)

        plsc.subcore_barrier()
        pltpu.sync_copy(acc.at[pl.ds(base, rpt)],
                        psums_h.at[cidx, pl.ds(base, rpt)])
        pltpu.sync_copy(cacc.at[pl.ds(base, rpt)], cntv.at[pl.ds(0, rpt)])
        pltpu.sync_copy(cntv.at[pl.ds(0, rpt)],
                        pcnts_h.at[pl.ds(cidx * NC2 + base, rpt)])

    mesh = plsc.VectorSubcoreMesh(core_axis_name="c", subcore_axis_name="s")
    fn = _sc_kernel(body, out_type=out_type, mesh=mesh, scratch_types=scratch)
    return fn(sums, cnts, pool_idx)


def _gather_rows(table, idx):
    n_chunks = N // CH
    trips = -(-n_chunks // NW)

    def body(table_h, idx_h, out_h, idxv, rows, sem):
        w = _worker_id()

        def trip(j, _):
            k = j * NW + w

            @pl.when(k < n_chunks)
            def _():
                off = k * CH
                pltpu.sync_copy(idx_h.at[pl.ds(off, CH)], idxv)
                pltpu.async_copy(table_h.at[idxv], rows, sem).wait()
                pltpu.sync_copy(rows, out_h.at[pl.ds(off, CH)])

            return 0

        lax.fori_loop(0, trips, trip, 0)

    mesh = plsc.VectorSubcoreMesh(core_axis_name="c", subcore_axis_name="s")
    fn = _sc_kernel(body,
                    out_type=jax.ShapeDtypeStruct((N, C), F32),
                    mesh=mesh,
                    scratch_types=(pltpu.VMEM((CH,), I32),
                                   pltpu.VMEM((CH, C), F32),
                                   pltpu.SemaphoreType.DMA))
    return fn(table, idx)


def _tc_call(fn, out_type, *args):
    return pl.pallas_call(fn, out_shape=out_type)(*args)


def _k_matmul(x, w):
    def body(x_ref, w_ref, o_ref):
        o_ref[...] = jnp.dot(x_ref[...], w_ref[...],
                             preferred_element_type=F32)

    return _tc_call(body, jax.ShapeDtypeStruct((x.shape[0], w.shape[1]), F32),
                    x, w)


def _k_mean_mm(sums, cnt3, w, relu):
    def body(s_ref, c_ref, w_ref, x_ref, y_ref):
        s = s_ref[0] + s_ref[1]
        d = jnp.maximum(c_ref[0] + c_ref[1], 1.0)
        x = s / d
        if relu:
            x = jnp.maximum(x, 0.0)
        x_ref[...] = x
        y_ref[...] = jnp.dot(x, w_ref[...], preferred_element_type=F32)

    out = (jax.ShapeDtypeStruct((NC2, C), F32),
           jax.ShapeDtypeStruct((NC2, C), F32))
    return _tc_call(body, out, sums, cnt3, w)


def _k_skip_merge(csums, cnt3, hc2, wm, pmat):
    def body(s_ref, c_ref, h_ref, wm_ref, p_ref, m_ref, ps_ref):
        d = jnp.maximum(c_ref[0] + c_ref[1], 1.0)
        s2 = (s_ref[0] + s_ref[1]) / d
        hc2 = h_ref[...]
        skip = jnp.maximum(s2 + hc2, 0.0)
        m_ref[...] = (jnp.dot(hc2, wm_ref[:C], preferred_element_type=F32)
                      + jnp.dot(skip, wm_ref[C:], preferred_element_type=F32))
        pa = jnp.dot(hc2, p_ref[...], preferred_element_type=F32)
        pb = jnp.dot(skip, p_ref[...], preferred_element_type=F32)
        ps_ref[...] = jnp.concatenate([pa, pb], axis=-1)

    out = (jax.ShapeDtypeStruct((NC2, C), F32),
           jax.ShapeDtypeStruct((NC2, C), F32))
    return _tc_call(body, out, csums, cnt3, hc2, wm, pmat)


def _k_final_mm(dsums, cnt3, pairsum, wup):
    def body(s_ref, c_ref, p_ref, w_ref, u_ref):
        d = jnp.maximum(c_ref[0] + c_ref[1], 1.0)
        merge = jnp.maximum((s_ref[0] + s_ref[1]) / d, 0.0)
        rf = merge + p_ref[...]
        u_ref[...] = jnp.maximum(
            jnp.dot(rf, w_ref[...], preferred_element_type=F32), 0.0)

    return _tc_call(body, jax.ShapeDtypeStruct((NC2, C), F32),
                    dsums, cnt3, pairsum, wup)


@jax.jit
def kernel(point_feat, edge_index, coarse_edge_index, pool_idx,
           W0, W1, Ws1, Ws2, Wm, Wup):
    EP = 322560
    ECP = 81920
    src = jnp.concatenate([edge_index[0], jnp.zeros((EP - E,), I32)])
    dst = jnp.concatenate(
        [edge_index[1], N + jnp.arange(EP - E, dtype=I32) % (N2 - N)])
    csrc = jnp.concatenate([coarse_edge_index[0],
                            jnp.zeros((ECP - EC,), I32)])
    cdst = jnp.concatenate(
        [coarse_edge_index[1],
         NC + jnp.arange(ECP - EC, dtype=I32) % (NC2 - NC)])
    pool_pad = jnp.concatenate(
        [pool_idx, NC + jnp.arange(N2 - N, dtype=I32) % (NC2 - NC)])
    pmat = jnp.repeat(jnp.eye(64, dtype=F32), 2, axis=0)

    p0 = _k_matmul(point_feat, W0)
    fsums, fcnts = _edge_pass_r1(p0, src, dst, N2, True)
    psums, pcnts = _pool_pass(fsums, fcnts, pool_pad)
    pcnt3 = pcnts.reshape(NCORES, NC2, 1)
    hc, h1 = _k_mean_mm(psums, pcnt3, W1, relu=False)
    asums, ccnts = _edge_pass_r1(h1, csrc, cdst, NC2, True)
    ccnt3 = ccnts.reshape(NCORES, NC2, 1)
    hc2, s1 = _k_mean_mm(asums, ccnt3, Ws1, relu=True)
    bsums = _edge_pass_r1(s1, csrc, cdst, NC2, False)[0]
    s, s2m = _k_mean_mm(bsums, ccnt3, Ws2, relu=True)
    csums = _edge_pass_r1(s2m, csrc, cdst, NC2, False)[0]
    m, pairsum = _k_skip_merge(csums, ccnt3, hc2, Wm, pmat)
    dsums = _edge_pass_r1(m, csrc, cdst, NC2, False)[0]
    u = _k_final_mm(dsums, ccnt3, pairsum, Wup)
    return _gather_rows(u, pool_idx)


# speedup vs baseline: 1.4426x; 1.4426x over previous
import functools

import jax
import jax.numpy as jnp
from jax import lax
from jax.experimental import pallas as pl
from jax.experimental.pallas import tpu as pltpu
from jax.experimental.pallas import tpu_sc as plsc

_sc_kernel = functools.partial(
    pl.kernel,
    compiler_params=pltpu.CompilerParams(needs_layout_passes=False))

N = 10000
N2 = 10240
NC = 2500
NC2 = 2560
E = 320000
EC = 80000
C = 128
NCORES = 2
NSUB = 16
NW = NCORES * NSUB
CH = 80
F32 = jnp.float32
I32 = jnp.int32


def _worker_id():
    return lax.axis_index("s") * NCORES + lax.axis_index("c")


def _zero_vmem_2d(ref, nrows):
    z = jnp.zeros((16,), F32)

    def body(i, _):
        ref[i // (C // 16), pl.ds((i % (C // 16)) * 16, 16)] = z
        return 0

    lax.fori_loop(0, nrows * (C // 16), body, 0)


def _zero_vmem_1d(ref, n):
    z = jnp.zeros((16,), F32)

    def body(i, _):
        ref[pl.ds(i * 16, 16)] = z
        return 0

    lax.fori_loop(0, n // 16, body, 0)


def _edge_pass(table, src, dst, acc_rows, with_counts, K):
    n_edges = dst.shape[0]
    n_chunks = n_edges // CH
    trips = n_chunks // NW
    assert n_edges % CH == 0 and n_chunks % NW == 0 and trips % K == 0
    rpt = acc_rows // NSUB
    nfull, rem = divmod(rpt, CH)
    assert rem == 0

    out_type = [jax.ShapeDtypeStruct((NCORES, acc_rows, C), F32)]
    scratch = [pltpu.VMEM_SHARED((acc_rows, C), F32)]
    scratch += [pltpu.VMEM((CH, C), F32) for _ in range(K)]
    scratch += [pltpu.VMEM((CH,), I32) for _ in range(2 * K)]
    scratch += [pltpu.SemaphoreType.DMA for _ in range(2 * K + 1)]
    if with_counts:
        out_type.append(jax.ShapeDtypeStruct((NCORES * acc_rows,), F32))
        out_type.append(jax.ShapeDtypeStruct((NW * acc_rows,), F32))
        scratch.append(pltpu.VMEM((acc_rows,), F32))
        scratch.append(pltpu.VMEM((NSUB * rpt,), F32))

    def body(*refs):
        it = iter(refs)
        table_h, src_h, dst_h = next(it), next(it), next(it)
        sums_h = next(it)
        cnts_h = next(it) if with_counts else None
        cparts_h = next(it) if with_counts else None
        acc = next(it)
        rows = [next(it) for _ in range(K)]
        srcv = [next(it) for _ in range(K)]
        dstv = [next(it) for _ in range(K)]
        isem = [next(it) for _ in range(K)]
        gsem = [next(it) for _ in range(K)]
        ssem = next(it)
        if with_counts:
            cntv, redv = next(it), next(it)

        cidx = lax.axis_index("c")
        sidx = lax.axis_index("s")
        w = _worker_id()

        _zero_vmem_2d(rows[0], CH)
        base = sidx * rpt
        for i in range(nfull):
            pltpu.sync_copy(rows[0], acc.at[pl.ds(base + i * CH, CH)])
        if with_counts:
            _zero_vmem_1d(cntv, acc_rows)
        plsc.subcore_barrier()

        ones16 = jnp.ones((16,), F32)

        def off_of(j):
            return (j * NW + w) * CH

        def do_counts(dv):
            if with_counts:
                for g in range(CH // 16):
                    d16 = dv[pl.ds(g * 16, 16)]
                    plsc.addupdate_scatter(cntv, (d16,), ones16)

        if K == 1:
            def trip(j, _):
                off = off_of(j)
                pltpu.sync_copy(src_h.at[pl.ds(off, CH)], srcv[0])
                pltpu.sync_copy(dst_h.at[pl.ds(off, CH)], dstv[0])
                pltpu.async_copy(table_h.at[srcv[0]], rows[0], gsem[0]).wait()
                pltpu.sync_copy(rows[0], acc.at[dstv[0]], add=True)
                do_counts(dstv[0])
                return 0

            lax.fori_loop(0, trips, trip, 0)

        def titer(t, _):
            j0 = t * K
            idescs = []
            for b in range(K):
                off = off_of(j0 + b)
                idescs.append(pltpu.async_copy(
                    src_h.at[pl.ds(off, CH)], srcv[b], isem[b]))
                idescs.append(pltpu.async_copy(
                    dst_h.at[pl.ds(off, CH)], dstv[b], isem[b]))
            gdescs = []
            for b in range(K):
                idescs[2 * b].wait()
                idescs[2 * b + 1].wait()
                gdescs.append(pltpu.async_copy(
                    table_h.at[srcv[b]], rows[b], gsem[b]))
            sdescs = []
            for b in range(K):
                gdescs[b].wait()
                sdescs.append(pltpu.async_copy(
                    rows[b], acc.at[dstv[b]], ssem, add=True))
                do_counts(dstv[b])
            for d in sdescs:
                d.wait()
            return 0

        if K > 1:
            lax.fori_loop(0, trips // K, titer, 0)

        if with_counts:
            pltpu.sync_copy(
                cntv,
                cparts_h.at[pl.ds((cidx * NSUB + sidx) * acc_rows, acc_rows)])
            plsc.subcore_barrier()
            rdescs = [pltpu.async_copy(
                cparts_h.at[pl.ds((cidx * NSUB + p) * acc_rows + base, rpt)],
                redv.at[pl.ds(p * rpt, rpt)], ssem) for p in range(NSUB)]
            for d in rdescs:
                d.wait()

            def redloop(i, _):
                tot = redv[pl.ds(i * 16, 16)]
                for p in range(1, NSUB):
                    tot = tot + redv[pl.ds(p * rpt + i * 16, 16)]
                cntv[pl.ds(i * 16, 16)] = tot
                return 0

            lax.fori_loop(0, rpt // 16, redloop, 0)
            pltpu.sync_copy(cntv.at[pl.ds(0, rpt)],
                            cnts_h.at[pl.ds(cidx * acc_rows + base, rpt)])

        plsc.subcore_barrier()
        pltpu.sync_copy(acc.at[pl.ds(base, rpt)],
                        sums_h.at[cidx, pl.ds(base, rpt)])

    mesh = plsc.VectorSubcoreMesh(core_axis_name="c", subcore_axis_name="s")
    fn = _sc_kernel(body, out_type=tuple(out_type), mesh=mesh,
                    scratch_types=tuple(scratch))
    return fn(table, src, dst)


def _edge_pass_r1(table, src, dst, acc_rows, with_counts):
    n_edges = dst.shape[0]
    assert n_edges % CH == 0
    n_chunks = n_edges // CH
    trips = -(-n_chunks // NW)
    rpt = acc_rows // NSUB
    nfull, rem = divmod(rpt, CH)

    out_type = [jax.ShapeDtypeStruct((NCORES, acc_rows, C), F32)]
    scratch = [
        pltpu.VMEM_SHARED((acc_rows, C), F32),
        pltpu.VMEM((CH, C), F32),
        pltpu.VMEM((CH,), I32),
        pltpu.VMEM((CH,), I32),
        pltpu.SemaphoreType.DMA,
    ]
    if with_counts:
        out_type.append(jax.ShapeDtypeStruct((NCORES * acc_rows,), F32))
        scratch.append(pltpu.VMEM_SHARED((acc_rows,), F32))
        scratch.append(pltpu.VMEM((acc_rows,), F32))
        scratch.append(pltpu.VMEM((CH,), I32))

    def body(*refs):
        it = iter(refs)
        table_h, src_h, dst_h = next(it), next(it), next(it)
        sums_h = next(it)
        cnts_h = next(it) if with_counts else None
        acc, rows, dstv, srcv, sem = (next(it), next(it), next(it), next(it),
                                      next(it))
        if with_counts:
            cacc, cntv, iotav = next(it), next(it), next(it)

        cidx = lax.axis_index("c")
        sidx = lax.axis_index("s")
        w = _worker_id()

        _zero_vmem_2d(rows, CH)
        base = sidx * rpt
        for i in range(nfull):
            pltpu.sync_copy(rows, acc.at[pl.ds(base + i * CH, CH)])
        if rem:
            pltpu.sync_copy(rows.at[pl.ds(0, rem)],
                            acc.at[pl.ds(base + nfull * CH, rem)])
        if with_counts:
            _zero_vmem_1d(cntv, acc_rows)
            for i in range(nfull):
                pltpu.sync_copy(cntv.at[pl.ds(0, CH)],
                                cacc.at[pl.ds(base + i * CH, CH)])
            if rem:
                pltpu.sync_copy(cntv.at[pl.ds(0, rem)],
                                cacc.at[pl.ds(base + nfull * CH, rem)])
        plsc.subcore_barrier()

        ones16 = jnp.ones((16,), F32)

        def trip(j, _):
            k = j * NW + w

            @pl.when(k < n_chunks)
            def _():
                off = k * CH
                pltpu.sync_copy(dst_h.at[pl.ds(off, CH)], dstv)
                pltpu.sync_copy(src_h.at[pl.ds(off, CH)], srcv)
                pltpu.async_copy(table_h.at[srcv], rows, sem).wait()
                pltpu.sync_copy(rows, acc.at[dstv], add=True)
                if with_counts:
                    for g in range(CH // 16):
                        d16 = dstv[pl.ds(g * 16, 16)]
                        plsc.addupdate_scatter(cntv, (d16,), ones16)

            return 0

        lax.fori_loop(0, trips, trip, 0)

        if with_counts:
            iota16 = lax.iota(I32, 16)
            for g in range(CH // 16):
                iotav[pl.ds(g * 16, 16)] = iota16 + g * 16

            def cmerge(j, _):
                pltpu.sync_copy(cntv.at[pl.ds(j * CH, CH)],
                                cacc.at[iotav], add=True)

                def bump(g, _):
                    v = iotav[pl.ds(g * 16, 16)]
                    iotav[pl.ds(g * 16, 16)] = v + CH
                    return 0

                lax.fori_loop(0, CH // 16, bump, 0)
                return 0

            lax.fori_loop(0, acc_rows // CH, cmerge, 0)

        plsc.subcore_barrier()
        pltpu.sync_copy(acc.at[pl.ds(base, rpt)],
                        sums_h.at[cidx, pl.ds(base, rpt)])
        if with_counts:
            pltpu.sync_copy(cacc.at[pl.ds(base, rpt)], cntv.at[pl.ds(0, rpt)])
            pltpu.sync_copy(cntv.at[pl.ds(0, rpt)],
                            cnts_h.at[pl.ds(cidx * acc_rows + base, rpt)])

    mesh = plsc.VectorSubcoreMesh(core_axis_name="c", subcore_axis_name="s")
    fn = _sc_kernel(body, out_type=tuple(out_type), mesh=mesh,
                    scratch_types=tuple(scratch))
    return fn(table, src, dst)


def _pool_pass(sums, cnts, pool_idx):
    n_chunks = N // CH
    trips = -(-n_chunks // NW)
    rpt = NC2 // NSUB
    nfull, rem = divmod(rpt, CH)

    out_type = (jax.ShapeDtypeStruct((NCORES, NC2, C), F32),
                jax.ShapeDtypeStruct((NCORES * NC2,), F32))
    scratch = (
        pltpu.VMEM_SHARED((NC2, C), F32),
        pltpu.VMEM_SHARED((NC2,), F32),
        pltpu.VMEM((CH, C), F32),
        pltpu.VMEM((CH, C), F32),
        pltpu.VMEM((CH, C), F32),
        pltpu.VMEM((CH,), F32),
        pltpu.VMEM((CH,), F32),
        pltpu.VMEM((CH,), I32),
        pltpu.VMEM((NC2,), F32),
        pltpu.VMEM((CH,), I32),
        pltpu.SemaphoreType.DMA,
    )

    def body(s_h, c_h, pool_h, psums_h, pcnts_h,
             acc, cacc, rows0, rows1, hbuf, c0v, c1v, dstv, cntv, iotav, sem):
        cidx = lax.axis_index("c")
        sidx = lax.axis_index("s")
        w = _worker_id()

        _zero_vmem_2d(hbuf, CH)
        base = sidx * rpt
        for i in range(nfull):
            pltpu.sync_copy(hbuf, acc.at[pl.ds(base + i * CH, CH)])
        if rem:
            pltpu.sync_copy(hbuf.at[pl.ds(0, rem)],
                            acc.at[pl.ds(base + nfull * CH, rem)])
        _zero_vmem_1d(cntv, NC2)
        for i in range(nfull):
            pltpu.sync_copy(cntv.at[pl.ds(0, CH)],
                            cacc.at[pl.ds(base + i * CH, CH)])
        if rem:
            pltpu.sync_copy(cntv.at[pl.ds(0, rem)],
                            cacc.at[pl.ds(base + nfull * CH, rem)])
        plsc.subcore_barrier()

        ones16 = jnp.ones((16,), F32)
        one16 = jnp.ones((16,), F32)

        def trip(j, _):
          k = j * NW + w

          @pl.when(k < n_chunks)
          def _():
            off = k * CH
            pltpu.sync_copy(s_h.at[0, pl.ds(off, CH)], rows0)
            pltpu.sync_copy(s_h.at[1, pl.ds(off, CH)], rows1)
            pltpu.sync_copy(c_h.at[pl.ds(off, CH)], c0v)
            pltpu.sync_copy(c_h.at[pl.ds(N2 + off, CH)], c1v)
            pltpu.sync_copy(pool_h.at[pl.ds(off, CH)], dstv)

            def row(r, _):
                ridx = jnp.full((16,), r, I32)
                d = (plsc.load_gather(c0v, (ridx,))
                     + plsc.load_gather(c1v, (ridx,)))
                rcp = one16 / jnp.maximum(d, 1.0)
                for f in range(C // 16):
                    v = (rows0[r, pl.ds(f * 16, 16)]
                         + rows1[r, pl.ds(f * 16, 16)]) * rcp
                    hbuf[r, pl.ds(f * 16, 16)] = jnp.maximum(v, 0.0)
                return 0

            lax.fori_loop(0, CH, row, 0)
            pltpu.sync_copy(hbuf, acc.at[dstv], add=True)
            for g in range(CH // 16):
                d16 = dstv[pl.ds(g * 16, 16)]
                plsc.addupdate_scatter(cntv, (d16,), ones16)

          return 0

        lax.fori_loop(0, trips, trip, 0)

        iota16 = lax.iota(I32, 16)
        for g in range(CH // 16):
            iotav[pl.ds(g * 16, 16)] = iota16 + g * 16

        def cmerge(j, _):
            pltpu.sync_copy(cntv.at[pl.ds(j * CH, CH)], cacc.at[iotav], add=True)

            def bump(g, _):
                v = iotav[pl.ds(g * 16, 16)]
                iotav[pl.ds(g * 16, 16)] = v + CH
                return 0

            lax.fori_loop(0, CH // 16, bump, 0)
            return 0

        lax.fori_loop(0, NC2 // CH, cmerge, 0)

        plsc.subcore_barrier()
        pltpu.sync_copy(acc.at[pl.ds(base, rpt)],
                        psums_h.at[cidx, pl.ds(base, rpt)])
        pltpu.sync_copy(cacc.at[pl.ds(base, rpt)], cntv.at[pl.ds(0, rpt)])
        pltpu.sync_copy(cntv.at[pl.ds(0, rpt)],
                        pcnts_h.at[pl.ds(cidx * NC2 + base, rpt)])

    mesh = plsc.VectorSubcoreMesh(core_axis_name="c", subcore_axis_name="s")
    fn = _sc_kernel(body, out_type=out_type, mesh=mesh, scratch_types=scratch)
    return fn(sums, cnts, pool_idx)


def _gather_rows(table, idx):
    n_chunks = N // CH
    trips = -(-n_chunks // NW)

    def body(table_h, idx_h, out_h, idxv, rows, sem):
        w = _worker_id()

        def trip(j, _):
            k = j * NW + w

            @pl.when(k < n_chunks)
            def _():
                off = k * CH
                pltpu.sync_copy(idx_h.at[pl.ds(off, CH)], idxv)
                pltpu.async_copy(table_h.at[idxv], rows, sem).wait()
                pltpu.sync_copy(rows, out_h.at[pl.ds(off, CH)])

            return 0

        lax.fori_loop(0, trips, trip, 0)

    mesh = plsc.VectorSubcoreMesh(core_axis_name="c", subcore_axis_name="s")
    fn = _sc_kernel(body,
                    out_type=jax.ShapeDtypeStruct((N, C), F32),
                    mesh=mesh,
                    scratch_types=(pltpu.VMEM((CH,), I32),
                                   pltpu.VMEM((CH, C), F32),
                                   pltpu.SemaphoreType.DMA))
    return fn(table, idx)


def _tc_call(fn, out_type, *args):
    return pl.pallas_call(fn, out_shape=out_type)(*args)


def _k_matmul(x, w):
    def body(x_ref, w_ref, o_ref):
        o_ref[...] = jnp.dot(x_ref[...], w_ref[...],
                             preferred_element_type=F32)

    return _tc_call(body, jax.ShapeDtypeStruct((x.shape[0], w.shape[1]), F32),
                    x, w)


def _k_mean_mm(sums, cnt3, w, relu):
    def body(s_ref, c_ref, w_ref, x_ref, y_ref):
        s = s_ref[0] + s_ref[1]
        d = jnp.maximum(c_ref[0] + c_ref[1], 1.0)
        x = s / d
        if relu:
            x = jnp.maximum(x, 0.0)
        x_ref[...] = x
        y_ref[...] = jnp.dot(x, w_ref[...], preferred_element_type=F32)

    out = (jax.ShapeDtypeStruct((NC2, C), F32),
           jax.ShapeDtypeStruct((NC2, C), F32))
    return _tc_call(body, out, sums, cnt3, w)


def _k_skip_merge(csums, cnt3, hc2, wm, pmat):
    def body(s_ref, c_ref, h_ref, wm_ref, p_ref, m_ref, ps_ref):
        d = jnp.maximum(c_ref[0] + c_ref[1], 1.0)
        s2 = (s_ref[0] + s_ref[1]) / d
        hc2 = h_ref[...]
        skip = jnp.maximum(s2 + hc2, 0.0)
        m_ref[...] = (jnp.dot(hc2, wm_ref[:C], preferred_element_type=F32)
                      + jnp.dot(skip, wm_ref[C:], preferred_element_type=F32))
        pa = jnp.dot(hc2, p_ref[...], preferred_element_type=F32)
        pb = jnp.dot(skip, p_ref[...], preferred_element_type=F32)
        ps_ref[...] = jnp.concatenate([pa, pb], axis=-1)

    out = (jax.ShapeDtypeStruct((NC2, C), F32),
           jax.ShapeDtypeStruct((NC2, C), F32))
    return _tc_call(body, out, csums, cnt3, hc2, wm, pmat)


def _k_final_mm(dsums, cnt3, pairsum, wup):
    def body(s_ref, c_ref, p_ref, w_ref, u_ref):
        d = jnp.maximum(c_ref[0] + c_ref[1], 1.0)
        merge = jnp.maximum((s_ref[0] + s_ref[1]) / d, 0.0)
        rf = merge + p_ref[...]
        u_ref[...] = jnp.maximum(
            jnp.dot(rf, w_ref[...], preferred_element_type=F32), 0.0)

    return _tc_call(body, jax.ShapeDtypeStruct((NC2, C), F32),
                    dsums, cnt3, pairsum, wup)


@jax.jit
def kernel(point_feat, edge_index, coarse_edge_index, pool_idx,
           W0, W1, Ws1, Ws2, Wm, Wup):
    src, dst = edge_index[0], edge_index[1]
    csrc, cdst = coarse_edge_index[0], coarse_edge_index[1]
    pool_pad = pool_idx
    pmat = jnp.repeat(jnp.eye(64, dtype=F32), 2, axis=0)

    p0 = _k_matmul(point_feat, W0)
    fsums, fcnts = _edge_pass_r1(p0, src, dst, N2, True)
    psums, pcnts = _pool_pass(fsums, fcnts, pool_pad)
    pcnt3 = pcnts.reshape(NCORES, NC2, 1)
    hc, h1 = _k_mean_mm(psums, pcnt3, W1, relu=False)
    asums, ccnts = _edge_pass_r1(h1, csrc, cdst, NC2, True)
    ccnt3 = ccnts.reshape(NCORES, NC2, 1)
    hc2, s1 = _k_mean_mm(asums, ccnt3, Ws1, relu=True)
    bsums = _edge_pass_r1(s1, csrc, cdst, NC2, False)[0]
    s, s2m = _k_mean_mm(bsums, ccnt3, Ws2, relu=True)
    csums = _edge_pass_r1(s2m, csrc, cdst, NC2, False)[0]
    m, pairsum = _k_skip_merge(csums, ccnt3, hc2, Wm, pmat)
    dsums = _edge_pass_r1(m, csrc, cdst, NC2, False)[0]
    u = _k_final_mm(dsums, ccnt3, pairsum, Wup)
    return _gather_rows(u, pool_idx)


# speedup vs baseline: 1.4635x; 1.0145x over previous
import functools

import jax
import jax.numpy as jnp
from jax import lax
from jax.experimental import pallas as pl
from jax.experimental.pallas import tpu as pltpu
from jax.experimental.pallas import tpu_sc as plsc

_sc_kernel = functools.partial(
    pl.kernel,
    compiler_params=pltpu.CompilerParams(needs_layout_passes=False))

N = 10000
N2 = 10240
NC = 2500
NC2 = 2560
E = 320000
EC = 80000
C = 128
NCORES = 2
NSUB = 16
NW = NCORES * NSUB
CH = 80
F32 = jnp.float32
I32 = jnp.int32


def _worker_id():
    return lax.axis_index("s") * NCORES + lax.axis_index("c")


def _zero_vmem_2d(ref, nrows):
    z = jnp.zeros((16,), F32)

    def body(i, _):
        ref[i // (C // 16), pl.ds((i % (C // 16)) * 16, 16)] = z
        return 0

    lax.fori_loop(0, nrows * (C // 16), body, 0)


def _zero_vmem_1d(ref, n):
    z = jnp.zeros((16,), F32)

    def body(i, _):
        ref[pl.ds(i * 16, 16)] = z
        return 0

    lax.fori_loop(0, n // 16, body, 0)


def _edge_pass(table, src, dst, acc_rows, with_counts, K):
    n_edges = dst.shape[0]
    n_chunks = n_edges // CH
    trips = n_chunks // NW
    assert n_edges % CH == 0 and n_chunks % NW == 0 and trips % K == 0
    rpt = acc_rows // NSUB
    nfull, rem = divmod(rpt, CH)
    assert rem == 0

    out_type = [jax.ShapeDtypeStruct((NCORES, acc_rows, C), F32)]
    scratch = [pltpu.VMEM_SHARED((acc_rows, C), F32)]
    scratch += [pltpu.VMEM((CH, C), F32) for _ in range(K)]
    scratch += [pltpu.VMEM((CH,), I32) for _ in range(2 * K)]
    scratch += [pltpu.SemaphoreType.DMA for _ in range(2 * K + 1)]
    if with_counts:
        out_type.append(jax.ShapeDtypeStruct((NCORES * acc_rows,), F32))
        out_type.append(jax.ShapeDtypeStruct((NW * acc_rows,), F32))
        scratch.append(pltpu.VMEM((acc_rows,), F32))
        scratch.append(pltpu.VMEM((NSUB * rpt,), F32))

    def body(*refs):
        it = iter(refs)
        table_h, src_h, dst_h = next(it), next(it), next(it)
        sums_h = next(it)
        cnts_h = next(it) if with_counts else None
        cparts_h = next(it) if with_counts else None
        acc = next(it)
        rows = [next(it) for _ in range(K)]
        srcv = [next(it) for _ in range(K)]
        dstv = [next(it) for _ in range(K)]
        isem = [next(it) for _ in range(K)]
        gsem = [next(it) for _ in range(K)]
        ssem = next(it)
        if with_counts:
            cntv, redv = next(it), next(it)

        cidx = lax.axis_index("c")
        sidx = lax.axis_index("s")
        w = _worker_id()

        _zero_vmem_2d(rows[0], CH)
        base = sidx * rpt
        for i in range(nfull):
            pltpu.sync_copy(rows[0], acc.at[pl.ds(base + i * CH, CH)])
        if with_counts:
            _zero_vmem_1d(cntv, acc_rows)
        plsc.subcore_barrier()

        ones16 = jnp.ones((16,), F32)

        def off_of(j):
            return (j * NW + w) * CH

        def do_counts(dv):
            if with_counts:
                for g in range(CH // 16):
                    d16 = dv[pl.ds(g * 16, 16)]
                    plsc.addupdate_scatter(cntv, (d16,), ones16)

        if K == 1:
            def trip(j, _):
                off = off_of(j)
                pltpu.sync_copy(src_h.at[pl.ds(off, CH)], srcv[0])
                pltpu.sync_copy(dst_h.at[pl.ds(off, CH)], dstv[0])
                pltpu.async_copy(table_h.at[srcv[0]], rows[0], gsem[0]).wait()
                pltpu.sync_copy(rows[0], acc.at[dstv[0]], add=True)
                do_counts(dstv[0])
                return 0

            lax.fori_loop(0, trips, trip, 0)

        def titer(t, _):
            j0 = t * K
            idescs = []
            for b in range(K):
                off = off_of(j0 + b)
                idescs.append(pltpu.async_copy(
                    src_h.at[pl.ds(off, CH)], srcv[b], isem[b]))
                idescs.append(pltpu.async_copy(
                    dst_h.at[pl.ds(off, CH)], dstv[b], isem[b]))
            gdescs = []
            for b in range(K):
                idescs[2 * b].wait()
                idescs[2 * b + 1].wait()
                gdescs.append(pltpu.async_copy(
                    table_h.at[srcv[b]], rows[b], gsem[b]))
            sdescs = []
            for b in range(K):
                gdescs[b].wait()
                sdescs.append(pltpu.async_copy(
                    rows[b], acc.at[dstv[b]], ssem, add=True))
                do_counts(dstv[b])
            for d in sdescs:
                d.wait()
            return 0

        if K > 1:
            lax.fori_loop(0, trips // K, titer, 0)

        if with_counts:
            pltpu.sync_copy(
                cntv,
                cparts_h.at[pl.ds((cidx * NSUB + sidx) * acc_rows, acc_rows)])
            plsc.subcore_barrier()
            rdescs = [pltpu.async_copy(
                cparts_h.at[pl.ds((cidx * NSUB + p) * acc_rows + base, rpt)],
                redv.at[pl.ds(p * rpt, rpt)], ssem) for p in range(NSUB)]
            for d in rdescs:
                d.wait()

            def redloop(i, _):
                tot = redv[pl.ds(i * 16, 16)]
                for p in range(1, NSUB):
                    tot = tot + redv[pl.ds(p * rpt + i * 16, 16)]
                cntv[pl.ds(i * 16, 16)] = tot
                return 0

            lax.fori_loop(0, rpt // 16, redloop, 0)
            pltpu.sync_copy(cntv.at[pl.ds(0, rpt)],
                            cnts_h.at[pl.ds(cidx * acc_rows + base, rpt)])

        plsc.subcore_barrier()
        pltpu.sync_copy(acc.at[pl.ds(base, rpt)],
                        sums_h.at[cidx, pl.ds(base, rpt)])

    mesh = plsc.VectorSubcoreMesh(core_axis_name="c", subcore_axis_name="s")
    fn = _sc_kernel(body, out_type=tuple(out_type), mesh=mesh,
                    scratch_types=tuple(scratch))
    return fn(table, src, dst)


def _edge_pass_r1(table, src, dst, acc_rows, with_counts):
    n_edges = dst.shape[0]
    assert n_edges % CH == 0
    n_chunks = n_edges // CH
    trips = -(-n_chunks // NW)
    rpt = acc_rows // NSUB
    nfull, rem = divmod(rpt, CH)

    out_type = [jax.ShapeDtypeStruct((NCORES, acc_rows, C), F32)]
    scratch = [
        pltpu.VMEM_SHARED((acc_rows, C), F32),
        pltpu.VMEM((CH, C), F32),
        pltpu.VMEM((CH,), I32),
        pltpu.VMEM((CH,), I32),
        pltpu.SemaphoreType.DMA,
    ]
    if with_counts:
        out_type.append(jax.ShapeDtypeStruct((NCORES * acc_rows,), F32))
        out_type.append(jax.ShapeDtypeStruct((NW * acc_rows,), F32))
        scratch.append(pltpu.VMEM((acc_rows,), F32))
        scratch.append(pltpu.VMEM((NSUB * rpt,), F32))

    def body(*refs):
        it = iter(refs)
        table_h, src_h, dst_h = next(it), next(it), next(it)
        sums_h = next(it)
        cnts_h = next(it) if with_counts else None
        cparts_h = next(it) if with_counts else None
        acc, rows, dstv, srcv, sem = (next(it), next(it), next(it), next(it),
                                      next(it))
        if with_counts:
            cntv, redv = next(it), next(it)

        cidx = lax.axis_index("c")
        sidx = lax.axis_index("s")
        w = _worker_id()

        _zero_vmem_2d(rows, CH)
        base = sidx * rpt
        for i in range(nfull):
            pltpu.sync_copy(rows, acc.at[pl.ds(base + i * CH, CH)])
        if rem:
            pltpu.sync_copy(rows.at[pl.ds(0, rem)],
                            acc.at[pl.ds(base + nfull * CH, rem)])
        if with_counts:
            _zero_vmem_1d(cntv, acc_rows)
        plsc.subcore_barrier()

        ones16 = jnp.ones((16,), F32)

        def trip(j, _):
            k = j * NW + w

            @pl.when(k < n_chunks)
            def _():
                off = k * CH
                pltpu.sync_copy(dst_h.at[pl.ds(off, CH)], dstv)
                pltpu.sync_copy(src_h.at[pl.ds(off, CH)], srcv)
                pltpu.async_copy(table_h.at[srcv], rows, sem).wait()
                pltpu.sync_copy(rows, acc.at[dstv], add=True)
                if with_counts:
                    for g in range(CH // 16):
                        d16 = dstv[pl.ds(g * 16, 16)]
                        plsc.addupdate_scatter(cntv, (d16,), ones16)

            return 0

        lax.fori_loop(0, trips, trip, 0)

        if with_counts:
            pltpu.sync_copy(
                cntv,
                cparts_h.at[pl.ds((cidx * NSUB + sidx) * acc_rows, acc_rows)])
            plsc.subcore_barrier()
            rdescs = [pltpu.async_copy(
                cparts_h.at[pl.ds((cidx * NSUB + q) * acc_rows + base, rpt)],
                redv.at[pl.ds(q * rpt, rpt)], sem) for q in range(NSUB)]
            for d in rdescs:
                d.wait()

            def redloop(i, _):
                tot = redv[pl.ds(i * 16, 16)]
                for q in range(1, NSUB):
                    tot = tot + redv[pl.ds(q * rpt + i * 16, 16)]
                cntv[pl.ds(i * 16, 16)] = tot
                return 0

            lax.fori_loop(0, rpt // 16, redloop, 0)
            pltpu.sync_copy(cntv.at[pl.ds(0, rpt)],
                            cnts_h.at[pl.ds(cidx * acc_rows + base, rpt)])

        plsc.subcore_barrier()
        pltpu.sync_copy(acc.at[pl.ds(base, rpt)],
                        sums_h.at[cidx, pl.ds(base, rpt)])

    mesh = plsc.VectorSubcoreMesh(core_axis_name="c", subcore_axis_name="s")
    fn = _sc_kernel(body, out_type=tuple(out_type), mesh=mesh,
                    scratch_types=tuple(scratch))
    return fn(table, src, dst)


def _pool_pass(sums, cnts, pool_idx):
    n_chunks = N // CH
    trips = -(-n_chunks // NW)
    rpt = NC2 // NSUB
    nfull, rem = divmod(rpt, CH)

    out_type = (jax.ShapeDtypeStruct((NCORES, NC2, C), F32),
                jax.ShapeDtypeStruct((NCORES * NC2,), F32))
    scratch = (
        pltpu.VMEM_SHARED((NC2, C), F32),
        pltpu.VMEM_SHARED((NC2,), F32),
        pltpu.VMEM((CH, C), F32),
        pltpu.VMEM((CH, C), F32),
        pltpu.VMEM((CH, C), F32),
        pltpu.VMEM((CH,), F32),
        pltpu.VMEM((CH,), F32),
        pltpu.VMEM((CH,), I32),
        pltpu.VMEM((NC2,), F32),
        pltpu.VMEM((CH,), I32),
        pltpu.SemaphoreType.DMA,
    )

    def body(s_h, c_h, pool_h, psums_h, pcnts_h,
             acc, cacc, rows0, rows1, hbuf, c0v, c1v, dstv, cntv, iotav, sem):
        cidx = lax.axis_index("c")
        sidx = lax.axis_index("s")
        w = _worker_id()

        _zero_vmem_2d(hbuf, CH)
        base = sidx * rpt
        for i in range(nfull):
            pltpu.sync_copy(hbuf, acc.at[pl.ds(base + i * CH, CH)])
        if rem:
            pltpu.sync_copy(hbuf.at[pl.ds(0, rem)],
                            acc.at[pl.ds(base + nfull * CH, rem)])
        _zero_vmem_1d(cntv, NC2)
        for i in range(nfull):
            pltpu.sync_copy(cntv.at[pl.ds(0, CH)],
                            cacc.at[pl.ds(base + i * CH, CH)])
        if rem:
            pltpu.sync_copy(cntv.at[pl.ds(0, rem)],
                            cacc.at[pl.ds(base + nfull * CH, rem)])
        plsc.subcore_barrier()

        ones16 = jnp.ones((16,), F32)
        one16 = jnp.ones((16,), F32)

        def trip(j, _):
          k = j * NW + w

          @pl.when(k < n_chunks)
          def _():
            off = k * CH
            pltpu.sync_copy(s_h.at[0, pl.ds(off, CH)], rows0)
            pltpu.sync_copy(s_h.at[1, pl.ds(off, CH)], rows1)
            pltpu.sync_copy(c_h.at[pl.ds(off, CH)], c0v)
            pltpu.sync_copy(c_h.at[pl.ds(N2 + off, CH)], c1v)
            pltpu.sync_copy(pool_h.at[pl.ds(off, CH)], dstv)

            def row(r, _):
                ridx = jnp.full((16,), r, I32)
                d = (plsc.load_gather(c0v, (ridx,))
                     + plsc.load_gather(c1v, (ridx,)))
                rcp = one16 / jnp.maximum(d, 1.0)
                for f in range(C // 16):
                    v = (rows0[r, pl.ds(f * 16, 16)]
                         + rows1[r, pl.ds(f * 16, 16)]) * rcp
                    hbuf[r, pl.ds(f * 16, 16)] = jnp.maximum(v, 0.0)
                return 0

            lax.fori_loop(0, CH, row, 0)
            pltpu.sync_copy(hbuf, acc.at[dstv], add=True)
            for g in range(CH // 16):
                d16 = dstv[pl.ds(g * 16, 16)]
                plsc.addupdate_scatter(cntv, (d16,), ones16)

          return 0

        lax.fori_loop(0, trips, trip, 0)

        iota16 = lax.iota(I32, 16)
        for g in range(CH // 16):
            iotav[pl.ds(g * 16, 16)] = iota16 + g * 16

        def cmerge(j, _):
            pltpu.sync_copy(cntv.at[pl.ds(j * CH, CH)], cacc.at[iotav], add=True)

            def bump(g, _):
                v = iotav[pl.ds(g * 16, 16)]
                iotav[pl.ds(g * 16, 16)] = v + CH
                return 0

            lax.fori_loop(0, CH // 16, bump, 0)
            return 0

        lax.fori_loop(0, NC2 // CH, cmerge, 0)

        plsc.subcore_barrier()
        pltpu.sync_copy(acc.at[pl.ds(base, rpt)],
                        psums_h.at[cidx, pl.ds(base, rpt)])
        pltpu.sync_copy(cacc.at[pl.ds(base, rpt)], cntv.at[pl.ds(0, rpt)])
        pltpu.sync_copy(cntv.at[pl.ds(0, rpt)],
                        pcnts_h.at[pl.ds(cidx * NC2 + base, rpt)])

    mesh = plsc.VectorSubcoreMesh(core_axis_name="c", subcore_axis_name="s")
    fn = _sc_kernel(body, out_type=out_type, mesh=mesh, scratch_types=scratch)
    return fn(sums, cnts, pool_idx)


def _gather_rows(table, idx):
    n_chunks = N // CH
    trips = -(-n_chunks // NW)

    def body(table_h, idx_h, out_h, idxv, rows, sem):
        w = _worker_id()

        def trip(j, _):
            k = j * NW + w

            @pl.when(k < n_chunks)
            def _():
                off = k * CH
                pltpu.sync_copy(idx_h.at[pl.ds(off, CH)], idxv)
                pltpu.async_copy(table_h.at[idxv], rows, sem).wait()
                pltpu.sync_copy(rows, out_h.at[pl.ds(off, CH)])

            return 0

        lax.fori_loop(0, trips, trip, 0)

    mesh = plsc.VectorSubcoreMesh(core_axis_name="c", subcore_axis_name="s")
    fn = _sc_kernel(body,
                    out_type=jax.ShapeDtypeStruct((N, C), F32),
                    mesh=mesh,
                    scratch_types=(pltpu.VMEM((CH,), I32),
                                   pltpu.VMEM((CH, C), F32),
                                   pltpu.SemaphoreType.DMA))
    return fn(table, idx)


def _tc_call(fn, out_type, *args):
    return pl.pallas_call(fn, out_shape=out_type)(*args)


def _k_matmul(x, w):
    def body(x_ref, w_ref, o_ref):
        o_ref[...] = jnp.dot(x_ref[...], w_ref[...],
                             preferred_element_type=F32)

    return _tc_call(body, jax.ShapeDtypeStruct((x.shape[0], w.shape[1]), F32),
                    x, w)


def _k_mean_mm(sums, cnt3, w, relu):
    def body(s_ref, c_ref, w_ref, x_ref, y_ref):
        s = s_ref[0] + s_ref[1]
        d = jnp.maximum(c_ref[0] + c_ref[1], 1.0)
        x = s / d
        if relu:
            x = jnp.maximum(x, 0.0)
        x_ref[...] = x
        y_ref[...] = jnp.dot(x, w_ref[...], preferred_element_type=F32)

    out = (jax.ShapeDtypeStruct((NC2, C), F32),
           jax.ShapeDtypeStruct((NC2, C), F32))
    return _tc_call(body, out, sums, cnt3, w)


def _k_skip_merge(csums, cnt3, hc2, wm, pmat):
    def body(s_ref, c_ref, h_ref, wm_ref, p_ref, m_ref, ps_ref):
        d = jnp.maximum(c_ref[0] + c_ref[1], 1.0)
        s2 = (s_ref[0] + s_ref[1]) / d
        hc2 = h_ref[...]
        skip = jnp.maximum(s2 + hc2, 0.0)
        m_ref[...] = (jnp.dot(hc2, wm_ref[:C], preferred_element_type=F32)
                      + jnp.dot(skip, wm_ref[C:], preferred_element_type=F32))
        pa = jnp.dot(hc2, p_ref[...], preferred_element_type=F32)
        pb = jnp.dot(skip, p_ref[...], preferred_element_type=F32)
        ps_ref[...] = jnp.concatenate([pa, pb], axis=-1)

    out = (jax.ShapeDtypeStruct((NC2, C), F32),
           jax.ShapeDtypeStruct((NC2, C), F32))
    return _tc_call(body, out, csums, cnt3, hc2, wm, pmat)


def _k_final_mm(dsums, cnt3, pairsum, wup):
    def body(s_ref, c_ref, p_ref, w_ref, u_ref):
        d = jnp.maximum(c_ref[0] + c_ref[1], 1.0)
        merge = jnp.maximum((s_ref[0] + s_ref[1]) / d, 0.0)
        rf = merge + p_ref[...]
        u_ref[...] = jnp.maximum(
            jnp.dot(rf, w_ref[...], preferred_element_type=F32), 0.0)

    return _tc_call(body, jax.ShapeDtypeStruct((NC2, C), F32),
                    dsums, cnt3, pairsum, wup)


@jax.jit
def kernel(point_feat, edge_index, coarse_edge_index, pool_idx,
           W0, W1, Ws1, Ws2, Wm, Wup):
    src, dst = edge_index[0], edge_index[1]
    csrc, cdst = coarse_edge_index[0], coarse_edge_index[1]
    pool_pad = pool_idx
    pmat = jnp.repeat(jnp.eye(64, dtype=F32), 2, axis=0)

    p0 = _k_matmul(point_feat, W0)
    fsums, fcnts, _ = _edge_pass_r1(p0, src, dst, N2, True)
    psums, pcnts = _pool_pass(fsums, fcnts, pool_pad)
    pcnt3 = pcnts.reshape(NCORES, NC2, 1)
    hc, h1 = _k_mean_mm(psums, pcnt3, W1, relu=False)
    asums, ccnts, _ = _edge_pass_r1(h1, csrc, cdst, NC2, True)
    ccnt3 = ccnts.reshape(NCORES, NC2, 1)
    hc2, s1 = _k_mean_mm(asums, ccnt3, Ws1, relu=True)
    bsums = _edge_pass_r1(s1, csrc, cdst, NC2, False)[0]
    s, s2m = _k_mean_mm(bsums, ccnt3, Ws2, relu=True)
    csums = _edge_pass_r1(s2m, csrc, cdst, NC2, False)[0]
    m, pairsum = _k_skip_merge(csums, ccnt3, hc2, Wm, pmat)
    dsums = _edge_pass_r1(m, csrc, cdst, NC2, False)[0]
    u = _k_final_mm(dsums, ccnt3, pairsum, Wup)
    return _gather_rows(u, pool_idx)
